# baseline (speedup 1.0000x reference)
import functools

import jax
import jax.numpy as jnp
from jax import lax
from jax.experimental import pallas as pl
from jax.experimental.pallas import tpu as pltpu
from jax.experimental.pallas import tpu_sc as plsc

N = 50000
D = 256
G = 256
NC = 2
NS = 16
DC = D // NC
NG16 = DC // 16
SBR = 64
BLK = 2000


def _sc_stats_body(x_hbm, batch_hbm, o1, o2, oc,
                   p_sh, pc_sh, s1v, s2v, cntv, xv, idxv):
    cid = lax.axis_index("c")
    sid = lax.axis_index("s")
    ccol = cid * DC
    ones16 = jnp.ones((16,), jnp.float32)
    zeros16 = jnp.zeros((16,), jnp.float32)

    def _zero(r, _):
        for c in range(NG16):
            s1v[r, pl.ds(c * 16, 16)] = zeros16
            s2v[r, pl.ds(c * 16, 16)] = zeros16
        cntv[r, :] = zeros16
        return 0

    lax.fori_loop(0, G, _zero, 0)

    start = sid * N // NS // 8 * 8
    end = (sid + 1) * N // NS // 8 * 8

    def _process(base, lo):
        pltpu.sync_copy(x_hbm.at[pl.ds(base, SBR), pl.ds(ccol, DC)], xv)
        pltpu.sync_copy(batch_hbm.at[pl.ds(base, SBR)], idxv)

        def _row16(i, _):
            idvec = idxv[pl.ds(i * 16, 16)]
            for l in range(16):
                r = i * 16 + l
                g = jnp.where(r < lo, G, idvec[l])
                for c in range(NG16):
                    v = xv[r, pl.ds(c * 16, 16)]
                    plsc.addupdate(s1v.at[g, pl.ds(c * 16, 16)], v)
                    plsc.addupdate(s2v.at[g, pl.ds(c * 16, 16)], v * v)
                plsc.addupdate(cntv.at[g], ones16)
            return 0

        lax.fori_loop(0, SBR // 16, _row16, 0)

    nrows = end - start
    nfull = nrows // SBR

    def _blk(k, _):
        _process(start + k * SBR, 0)
        return 0

    lax.fori_loop(0, nfull, _blk, 0)
    rem = nrows - nfull * SBR

    @pl.when(rem > 0)
    def _tail():
        _process(end - SBR, SBR - rem)

    GH = G // 2
    STRIP = GH // NS

    def _phase(src_v, glo, out_hbm):
        pltpu.sync_copy(src_v.at[pl.ds(glo, GH)], p_sh.at[sid])
        plsc.subcore_barrier()
        rlo = sid * STRIP
        pltpu.sync_copy(p_sh.at[0, pl.ds(rlo, STRIP)], xv.at[pl.ds(0, STRIP)])

        def _racc(j, _):
            pltpu.sync_copy(p_sh.at[j, pl.ds(rlo, STRIP)],
                            xv.at[pl.ds(STRIP, STRIP)])

            def _radd(r, _):
                for c in range(NG16):
                    plsc.addupdate(xv.at[r, pl.ds(c * 16, 16)],
                                   xv[STRIP + r, pl.ds(c * 16, 16)])
                return 0

            lax.fori_loop(0, STRIP, _radd, 0)
            return 0

        lax.fori_loop(1, NS, _racc, 0)
        pltpu.sync_copy(xv.at[pl.ds(0, STRIP)],
                        out_hbm.at[pl.ds(glo + rlo, STRIP), pl.ds(ccol, DC)])
        plsc.subcore_barrier()

    pltpu.sync_copy(cntv.at[pl.ds(0, G)], pc_sh.at[sid])
    _phase(s1v, 0, o1)
    _phase(s1v, GH, o1)
    _phase(s2v, 0, o2)
    _phase(s2v, GH, o2)

    rlo16 = sid * (G // NS)
    pltpu.sync_copy(pc_sh.at[0, pl.ds(rlo16, 16)], cntv.at[pl.ds(0, 16)])

    def _cacc(j, _):
        pltpu.sync_copy(pc_sh.at[j, pl.ds(rlo16, 16)], cntv.at[pl.ds(16, 16)])

        def _cadd(r, _):
            plsc.addupdate(cntv.at[r], cntv[16 + r, :])
            return 0

        lax.fori_loop(0, 16, _cadd, 0)
        return 0

    lax.fori_loop(1, NS, _cacc, 0)

    @pl.when(cid == 0)
    def _wcnt():
        pltpu.sync_copy(cntv.at[pl.ds(0, 16)], oc.at[pl.ds(rlo16, 16)])


def _finalize_body(o1_ref, o2_ref, oc_ref, w_ref, b_ref, s_ref, rb_ref):
    s1 = o1_ref[...]
    s2 = o2_ref[...]
    cnt = oc_ref[:, :1]
    denom = jnp.maximum(cnt, 1.0)
    a = (s1 / denom) * s_ref[...]
    var = (s2 - 2.0 * a * s1 + cnt * a * a) / denom
    r = w_ref[...] * lax.rsqrt(var + 1e-8)
    bp = b_ref[...] - a * r
    rb_ref[...] = jnp.concatenate([r, bp], axis=1)


def _norm_body(x_ref, batch_ref, rb_ref, out_ref):
    bb = batch_ref[0, 0, :]
    onehot = (bb[:, None] == lax.broadcasted_iota(jnp.int32, (BLK, G), 1)
              ).astype(jnp.float32)
    g = lax.dot_general(onehot, rb_ref[...], (((1,), (0,)), ((), ())),
                        preferred_element_type=jnp.float32)
    x = x_ref[...]
    out_ref[...] = x * g[:, :D] + g[:, D:]


@jax.jit
def kernel(node_emb, weight, bias, scale, batch):
    n, d = node_emb.shape
    nb = n // BLK
    batch_i = batch.astype(jnp.int32)

    mesh = plsc.VectorSubcoreMesh(core_axis_name="c", subcore_axis_name="s")
    o1, o2, oc = pl.kernel(
        _sc_stats_body,
        out_type=(
            jax.ShapeDtypeStruct((G, D), jnp.float32),
            jax.ShapeDtypeStruct((G, D), jnp.float32),
            jax.ShapeDtypeStruct((G, 16), jnp.float32),
        ),
        mesh=mesh,
        scratch_types=[
            pltpu.VMEM_SHARED((NS, G // 2, DC), jnp.float32),
            pltpu.VMEM_SHARED((NS, G, 16), jnp.float32),
            pltpu.VMEM((G + 8, DC), jnp.float32),
            pltpu.VMEM((G + 8, DC), jnp.float32),
            pltpu.VMEM((G + 8, 16), jnp.float32),
            pltpu.VMEM((SBR, DC), jnp.float32),
            pltpu.VMEM((SBR,), jnp.int32),
        ],
    )(node_emb, batch_i)

    w2 = weight.reshape(1, d)
    b2 = bias.reshape(1, d)
    s2 = scale.reshape(1, d)
    rb = pl.pallas_call(
        _finalize_body,
        out_shape=jax.ShapeDtypeStruct((G, 2 * d), jnp.float32),
    )(o1, o2, oc, w2, b2, s2)

    batch3 = batch_i.reshape(nb, 1, BLK)
    out = pl.pallas_call(
        _norm_body,
        grid=(nb,),
        in_specs=[
            pl.BlockSpec((BLK, d), lambda i: (i, 0)),
            pl.BlockSpec((1, 1, BLK), lambda i: (i, 0, 0)),
            pl.BlockSpec((G, 2 * d), lambda i: (0, 0)),
        ],
        out_specs=pl.BlockSpec((BLK, d), lambda i: (i, 0)),
        out_shape=jax.ShapeDtypeStruct((n, d), jnp.float32),
    )(node_emb, batch3, rb)
    return out
